# baseline (speedup 1.0000x reference)
import functools

import jax
import jax.numpy as jnp
from jax import lax
from jax.experimental import pallas as pl
from jax.experimental.pallas import tpu as pltpu
from jax.experimental.pallas import tpu_sc as plsc

_IN_DIM = 256
_OUT_DIM = 512
_MEM_LEN = 100000
_BETA = 1.0
_BLK_A = 2000
_BLK_B = 4000

_NC = 2
_NS = 16
_NW = _NC * _NS
_TILES = _MEM_LEN // 8
_TQ = _TILES // _NW
_TR = _TILES - _TQ * _NW


def _sc_copy_body(src_hbm, dst_hbm, sem0, sem1):
    wid = lax.axis_index("s") * _NC + lax.axis_index("c")
    tbase = wid * _TQ + jnp.minimum(wid, _TR)
    base = 8 * tbase
    h0 = pltpu.async_copy(
        src_hbm.at[pl.ds(base, 8 * _TQ)], dst_hbm.at[pl.ds(base, 8 * _TQ)],
        sem0)

    @pl.when(wid < _TR)
    def _():
        extra = base + 8 * _TQ
        pltpu.async_copy(
            src_hbm.at[pl.ds(extra, 8)], dst_hbm.at[pl.ds(extra, 8)],
            sem1).wait()

    h0.wait()


_sc_copy = functools.partial(
    pl.kernel,
    mesh=plsc.VectorSubcoreMesh(core_axis_name="c", subcore_axis_name="s"),
    out_type=jax.ShapeDtypeStruct((_MEM_LEN, _OUT_DIM), jnp.float32),
    scratch_types=[
        pltpu.SemaphoreType.DMA,
        pltpu.SemaphoreType.DMA,
    ],
)(_sc_copy_body)


def _pass_a_body(in_ref, out_ref, sum_ref, sumsq_ref, acc_s, acc_q):
    i = pl.program_id(0)
    blk = in_ref[...]
    out_ref[...] = blk
    s = jnp.sum(blk, axis=0, keepdims=True)
    q = jnp.sum(blk * blk, axis=0, keepdims=True)

    @pl.when(i == 0)
    def _():
        acc_s[...] = s
        acc_q[...] = q

    @pl.when(i > 0)
    def _():
        acc_s[...] = acc_s[...] + s
        acc_q[...] = acc_q[...] + q

    @pl.when(i == pl.num_programs(0) - 1)
    def _():
        sum_ref[...] = acc_s[...]
        sumsq_ref[...] = acc_q[...]


def _mlp_body(x_ref, s_ref, q_ref, w1, b1, w2, b2, w3, b3, enc_ref):
    n = jnp.float32(_MEM_LEN)
    s = s_ref[...]
    q = q_ref[...]
    mean = s / n
    var = (q - s * (s / n)) / (n - 1.0)
    std = jnp.sqrt(var)
    xn = (x_ref[...] - mean) / std
    xn = jnp.where(std == 0.0, 0.0, xn)
    h1 = jnp.maximum(
        jnp.dot(xn, w1[...], preferred_element_type=jnp.float32) + b1[...], 0.0)
    h2 = jnp.maximum(
        jnp.dot(h1, w2[...], preferred_element_type=jnp.float32) + b2[...], 0.0)
    enc_ref[...] = jnp.tanh(
        jnp.dot(h2, w3[...], preferred_element_type=jnp.float32) + b3[...])


def _dist_body(mem_ref, enc_ref, loss_ref, min_s):
    i = pl.program_id(0)
    m = jnp.min(jnp.sum(jnp.abs(mem_ref[...] - enc_ref[...]), axis=1))

    @pl.when(i == 0)
    def _():
        min_s[0] = m

    @pl.when(i > 0)
    def _():
        min_s[0] = jnp.minimum(min_s[0], m)

    @pl.when(i == pl.num_programs(0) - 1)
    def _():
        loss_ref[0, 0] = min_s[0]


def _fixup_body(mem_in, md_in, loss_ref, enc_ref, x_ref, mem_out, md_out):
    mem_out[...] = mem_in[...]
    md_out[...] = md_in[...]

    @pl.when(loss_ref[0, 0] <= _BETA)
    def _():
        mem_out[0:1, :] = enc_ref[...]
        md_out[0:1, :] = x_ref[...]


def kernel(x, mem_data, memory, W1, b1, W2, b2, W3, b3):
    f32 = jnp.float32
    W1p = jnp.pad(W1, ((0, 0), (0, 12)))
    b1p = jnp.pad(b1, (0, 12)).reshape(1, 512)
    W2p = jnp.pad(W2, ((0, 12), (0, 24)))
    b2p = jnp.pad(b2, (0, 24)).reshape(1, 1024)
    W3p = jnp.pad(W3, ((0, 24), (0, 0)))
    b3p = b3.reshape(1, 512)

    new_memory = _sc_copy(memory)

    na = _MEM_LEN // _BLK_A
    new_mem_data, col_sum, col_sumsq = pl.pallas_call(
        _pass_a_body,
        grid=(na,),
        in_specs=[pl.BlockSpec((_BLK_A, _IN_DIM), lambda i: (i, 0))],
        out_specs=[
            pl.BlockSpec((_BLK_A, _IN_DIM), lambda i: (i, 0)),
            pl.BlockSpec((1, _IN_DIM), lambda i: (0, 0)),
            pl.BlockSpec((1, _IN_DIM), lambda i: (0, 0)),
        ],
        out_shape=[
            jax.ShapeDtypeStruct((_MEM_LEN, _IN_DIM), f32),
            jax.ShapeDtypeStruct((1, _IN_DIM), f32),
            jax.ShapeDtypeStruct((1, _IN_DIM), f32),
        ],
        scratch_shapes=[
            pltpu.VMEM((1, _IN_DIM), f32),
            pltpu.VMEM((1, _IN_DIM), f32),
        ],
    )(mem_data)

    enc = pl.pallas_call(
        _mlp_body,
        out_shape=jax.ShapeDtypeStruct((1, _OUT_DIM), f32),
    )(x, col_sum, col_sumsq, W1p, b1p, W2p, b2p, W3p, b3p)

    nb = _MEM_LEN // _BLK_B
    loss11 = pl.pallas_call(
        _dist_body,
        grid=(nb,),
        in_specs=[
            pl.BlockSpec((_BLK_B, _OUT_DIM), lambda i: (i, 0)),
            pl.BlockSpec((1, _OUT_DIM), lambda i: (0, 0)),
        ],
        out_specs=pl.BlockSpec(memory_space=pltpu.SMEM),
        out_shape=jax.ShapeDtypeStruct((1, 1), f32),
        scratch_shapes=[pltpu.SMEM((1,), f32)],
    )(memory, enc)

    new_memory, new_mem_data = pl.pallas_call(
        _fixup_body,
        grid=(1,),
        in_specs=[
            pl.BlockSpec((8, _OUT_DIM), lambda i: (0, 0)),
            pl.BlockSpec((8, _IN_DIM), lambda i: (0, 0)),
            pl.BlockSpec(memory_space=pltpu.SMEM),
            pl.BlockSpec((1, _OUT_DIM), lambda i: (0, 0)),
            pl.BlockSpec((1, _IN_DIM), lambda i: (0, 0)),
        ],
        out_specs=[
            pl.BlockSpec((8, _OUT_DIM), lambda i: (0, 0)),
            pl.BlockSpec((8, _IN_DIM), lambda i: (0, 0)),
        ],
        out_shape=[
            jax.ShapeDtypeStruct((_MEM_LEN, _OUT_DIM), f32),
            jax.ShapeDtypeStruct((_MEM_LEN, _IN_DIM), f32),
        ],
        input_output_aliases={0: 0, 1: 1},
    )(new_memory, new_mem_data, loss11, enc, x)

    return loss11[0, 0], new_memory, new_mem_data


# speedup vs baseline: 20.7637x; 20.7637x over previous
import functools

import jax
import jax.numpy as jnp
from jax import lax
from jax.experimental import pallas as pl
from jax.experimental.pallas import tpu as pltpu
from jax.experimental.pallas import tpu_sc as plsc

_IN_DIM = 256
_OUT_DIM = 512
_MEM_LEN = 100000
_BETA = 1.0
_BLK_A = 2000
_BLK_B = 4000

_NC = 2
_NS = 16
_NW = _NC * _NS
_TILES = _MEM_LEN // 8
_TQ = _TILES // _NW
_TR = _TILES - _TQ * _NW


_CC = 120
_NCC = 8 * _TQ // _CC


def _sc_copy_body(src_hbm, dst_hbm, buf0, buf1, sin0, sin1, sout0, sout1,
                  sem_x):
    wid = lax.axis_index("s") * _NC + lax.axis_index("c")
    base = 8 * (wid * _TQ + jnp.minimum(wid, _TR))
    bufs = (buf0, buf1)
    sins = (sin0, sin1)
    souts = (sout0, sout1)
    in_h = [None, None]
    out_h = [None, None]
    in_h[0] = pltpu.async_copy(src_hbm.at[pl.ds(base, _CC)], buf0, sin0)
    for k in range(_NCC):
        b = k % 2
        nb = (k + 1) % 2
        if k + 1 < _NCC:
            if k >= 1:
                out_h[nb].wait()
            in_h[nb] = pltpu.async_copy(
                src_hbm.at[pl.ds(base + (k + 1) * _CC, _CC)], bufs[nb],
                sins[nb])
        in_h[b].wait()
        out_h[b] = pltpu.async_copy(
            bufs[b], dst_hbm.at[pl.ds(base + k * _CC, _CC)], souts[b])

    out_h[(_NCC - 2) % 2].wait()
    out_h[(_NCC - 1) % 2].wait()

    @pl.when(wid < _TR)
    def _():
        extra = base + 8 * _TQ
        pltpu.async_copy(
            src_hbm.at[pl.ds(extra, 8)], buf0.at[pl.ds(0, 8)], sem_x).wait()
        pltpu.async_copy(
            buf0.at[pl.ds(0, 8)], dst_hbm.at[pl.ds(extra, 8)], sem_x).wait()


_sc_copy = functools.partial(
    pl.kernel,
    mesh=plsc.VectorSubcoreMesh(core_axis_name="c", subcore_axis_name="s"),
    out_type=jax.ShapeDtypeStruct((_MEM_LEN, _OUT_DIM), jnp.float32),
    scratch_types=[
        pltpu.VMEM((_CC, _OUT_DIM), jnp.float32),
        pltpu.VMEM((_CC, _OUT_DIM), jnp.float32),
        pltpu.SemaphoreType.DMA,
        pltpu.SemaphoreType.DMA,
        pltpu.SemaphoreType.DMA,
        pltpu.SemaphoreType.DMA,
        pltpu.SemaphoreType.DMA,
    ],
)(_sc_copy_body)


def _pass_a_body(in_ref, out_ref, sum_ref, sumsq_ref, acc_s, acc_q):
    i = pl.program_id(0)
    blk = in_ref[...]
    out_ref[...] = blk
    s = jnp.sum(blk, axis=0, keepdims=True)
    q = jnp.sum(blk * blk, axis=0, keepdims=True)

    @pl.when(i == 0)
    def _():
        acc_s[...] = s
        acc_q[...] = q

    @pl.when(i > 0)
    def _():
        acc_s[...] = acc_s[...] + s
        acc_q[...] = acc_q[...] + q

    @pl.when(i == pl.num_programs(0) - 1)
    def _():
        sum_ref[...] = acc_s[...]
        sumsq_ref[...] = acc_q[...]


def _mlp_body(x_ref, s_ref, q_ref, w1, b1, w2, b2, w3, b3, enc_ref):
    n = jnp.float32(_MEM_LEN)
    s = s_ref[...]
    q = q_ref[...]
    mean = s / n
    var = (q - s * (s / n)) / (n - 1.0)
    std = jnp.sqrt(var)
    xn = (x_ref[...] - mean) / std
    xn = jnp.where(std == 0.0, 0.0, xn)
    h1 = jnp.maximum(
        jnp.dot(xn, w1[...], preferred_element_type=jnp.float32) + b1[...], 0.0)
    h2 = jnp.maximum(
        jnp.dot(h1, w2[...], preferred_element_type=jnp.float32) + b2[...], 0.0)
    enc_ref[...] = jnp.tanh(
        jnp.dot(h2, w3[...], preferred_element_type=jnp.float32) + b3[...])


def _dist_body(mem_ref, enc_ref, loss_ref, min_s):
    i = pl.program_id(0)
    m = jnp.min(jnp.sum(jnp.abs(mem_ref[...] - enc_ref[...]), axis=1))

    @pl.when(i == 0)
    def _():
        min_s[0] = m

    @pl.when(i > 0)
    def _():
        min_s[0] = jnp.minimum(min_s[0], m)

    @pl.when(i == pl.num_programs(0) - 1)
    def _():
        loss_ref[0, 0] = min_s[0]


def _fixup_body(mem_in, md_in, loss_ref, enc_ref, x_ref, mem_out, md_out):
    mem_out[...] = mem_in[...]
    md_out[...] = md_in[...]

    @pl.when(loss_ref[0, 0] <= _BETA)
    def _():
        mem_out[0:1, :] = enc_ref[...]
        md_out[0:1, :] = x_ref[...]


def kernel(x, mem_data, memory, W1, b1, W2, b2, W3, b3):
    f32 = jnp.float32
    W1p = jnp.pad(W1, ((0, 0), (0, 12)))
    b1p = jnp.pad(b1, (0, 12)).reshape(1, 512)
    W2p = jnp.pad(W2, ((0, 12), (0, 24)))
    b2p = jnp.pad(b2, (0, 24)).reshape(1, 1024)
    W3p = jnp.pad(W3, ((0, 24), (0, 0)))
    b3p = b3.reshape(1, 512)

    new_memory = _sc_copy(memory)

    na = _MEM_LEN // _BLK_A
    new_mem_data, col_sum, col_sumsq = pl.pallas_call(
        _pass_a_body,
        grid=(na,),
        in_specs=[pl.BlockSpec((_BLK_A, _IN_DIM), lambda i: (i, 0))],
        out_specs=[
            pl.BlockSpec((_BLK_A, _IN_DIM), lambda i: (i, 0)),
            pl.BlockSpec((1, _IN_DIM), lambda i: (0, 0)),
            pl.BlockSpec((1, _IN_DIM), lambda i: (0, 0)),
        ],
        out_shape=[
            jax.ShapeDtypeStruct((_MEM_LEN, _IN_DIM), f32),
            jax.ShapeDtypeStruct((1, _IN_DIM), f32),
            jax.ShapeDtypeStruct((1, _IN_DIM), f32),
        ],
        scratch_shapes=[
            pltpu.VMEM((1, _IN_DIM), f32),
            pltpu.VMEM((1, _IN_DIM), f32),
        ],
    )(mem_data)

    enc = pl.pallas_call(
        _mlp_body,
        out_shape=jax.ShapeDtypeStruct((1, _OUT_DIM), f32),
    )(x, col_sum, col_sumsq, W1p, b1p, W2p, b2p, W3p, b3p)

    nb = _MEM_LEN // _BLK_B
    loss11 = pl.pallas_call(
        _dist_body,
        grid=(nb,),
        in_specs=[
            pl.BlockSpec((_BLK_B, _OUT_DIM), lambda i: (i, 0)),
            pl.BlockSpec((1, _OUT_DIM), lambda i: (0, 0)),
        ],
        out_specs=pl.BlockSpec(memory_space=pltpu.SMEM),
        out_shape=jax.ShapeDtypeStruct((1, 1), f32),
        scratch_shapes=[pltpu.SMEM((1,), f32)],
    )(memory, enc)

    new_memory, new_mem_data = pl.pallas_call(
        _fixup_body,
        grid=(1,),
        in_specs=[
            pl.BlockSpec((8, _OUT_DIM), lambda i: (0, 0)),
            pl.BlockSpec((8, _IN_DIM), lambda i: (0, 0)),
            pl.BlockSpec(memory_space=pltpu.SMEM),
            pl.BlockSpec((1, _OUT_DIM), lambda i: (0, 0)),
            pl.BlockSpec((1, _IN_DIM), lambda i: (0, 0)),
        ],
        out_specs=[
            pl.BlockSpec((8, _OUT_DIM), lambda i: (0, 0)),
            pl.BlockSpec((8, _IN_DIM), lambda i: (0, 0)),
        ],
        out_shape=[
            jax.ShapeDtypeStruct((_MEM_LEN, _OUT_DIM), f32),
            jax.ShapeDtypeStruct((_MEM_LEN, _IN_DIM), f32),
        ],
        input_output_aliases={0: 0, 1: 1},
    )(new_memory, new_mem_data, loss11, enc, x)

    return loss11[0, 0], new_memory, new_mem_data


# speedup vs baseline: 28.7812x; 1.3861x over previous
import jax
import jax.numpy as jnp
from jax.experimental import pallas as pl
from jax.experimental.pallas import tpu as pltpu

_IN_DIM = 256
_OUT_DIM = 512
_MEM_LEN = 100000
_BETA = 1.0
_BLK_A = 4000
_BLK_B = 4000


def _pass_a_body(in_ref, out_ref, sum_ref, sumsq_ref, acc_s, acc_q):
    i = pl.program_id(0)
    blk = in_ref[...]
    out_ref[...] = blk
    s = jnp.sum(blk, axis=0, keepdims=True)
    q = jnp.sum(blk * blk, axis=0, keepdims=True)

    @pl.when(i == 0)
    def _():
        acc_s[...] = s
        acc_q[...] = q

    @pl.when(i > 0)
    def _():
        acc_s[...] = acc_s[...] + s
        acc_q[...] = acc_q[...] + q

    @pl.when(i == pl.num_programs(0) - 1)
    def _():
        sum_ref[...] = acc_s[...]
        sumsq_ref[...] = acc_q[...]


def _mlp_body(x_ref, s_ref, q_ref, w1, b1, w2, b2, w3, b3, enc_ref):
    n = jnp.float32(_MEM_LEN)
    s = s_ref[...]
    q = q_ref[...]
    mean = s / n
    var = (q - s * (s / n)) / (n - 1.0)
    std = jnp.sqrt(var)
    xn = (x_ref[...] - mean) / std
    xn = jnp.where(std == 0.0, 0.0, xn)
    h1 = jnp.maximum(
        jnp.dot(xn, w1[...], preferred_element_type=jnp.float32) + b1[...], 0.0)
    h2 = jnp.maximum(
        jnp.dot(h1, w2[...], preferred_element_type=jnp.float32) + b2[...], 0.0)
    enc_ref[...] = jnp.tanh(
        jnp.dot(h2, w3[...], preferred_element_type=jnp.float32) + b3[...])


def _pass_b_body(mem_ref, enc_ref, x_ref, md_in, out_ref, loss_ref, md_out,
                 min_s):
    i = pl.program_id(0)
    blk = mem_ref[...]
    out_ref[...] = blk
    m = jnp.min(jnp.sum(jnp.abs(blk - enc_ref[...]), axis=1))

    @pl.when(i == 0)
    def _():
        min_s[0] = m

    @pl.when(i > 0)
    def _():
        min_s[0] = jnp.minimum(min_s[0], m)

    @pl.when(i == pl.num_programs(0) - 1)
    def _():
        loss = min_s[0]
        loss_ref[0, 0] = loss
        md_out[...] = md_in[...]

        @pl.when(loss <= _BETA)
        def _():
            out_ref[0:1, :] = enc_ref[...]
            md_out[0:1, :] = x_ref[...]


def kernel(x, mem_data, memory, W1, b1, W2, b2, W3, b3):
    f32 = jnp.float32
    W1p = jnp.pad(W1, ((0, 0), (0, 12)))
    b1p = jnp.pad(b1, (0, 12)).reshape(1, 512)
    W2p = jnp.pad(W2, ((0, 12), (0, 24)))
    b2p = jnp.pad(b2, (0, 24)).reshape(1, 1024)
    W3p = jnp.pad(W3, ((0, 24), (0, 0)))
    b3p = b3.reshape(1, 512)

    na = _MEM_LEN // _BLK_A
    new_mem_data, col_sum, col_sumsq = pl.pallas_call(
        _pass_a_body,
        grid=(na,),
        in_specs=[pl.BlockSpec((_BLK_A, _IN_DIM), lambda i: (i, 0))],
        out_specs=[
            pl.BlockSpec((_BLK_A, _IN_DIM), lambda i: (i, 0)),
            pl.BlockSpec((1, _IN_DIM), lambda i: (0, 0)),
            pl.BlockSpec((1, _IN_DIM), lambda i: (0, 0)),
        ],
        out_shape=[
            jax.ShapeDtypeStruct((_MEM_LEN, _IN_DIM), f32),
            jax.ShapeDtypeStruct((1, _IN_DIM), f32),
            jax.ShapeDtypeStruct((1, _IN_DIM), f32),
        ],
        scratch_shapes=[
            pltpu.VMEM((1, _IN_DIM), f32),
            pltpu.VMEM((1, _IN_DIM), f32),
        ],
    )(mem_data)

    enc = pl.pallas_call(
        _mlp_body,
        out_shape=jax.ShapeDtypeStruct((1, _OUT_DIM), f32),
    )(x, col_sum, col_sumsq, W1p, b1p, W2p, b2p, W3p, b3p)

    nb = _MEM_LEN // _BLK_B
    new_memory, loss11, new_mem_data = pl.pallas_call(
        _pass_b_body,
        grid=(nb,),
        in_specs=[
            pl.BlockSpec((_BLK_B, _OUT_DIM), lambda i, nb=nb: (nb - 1 - i, 0)),
            pl.BlockSpec((1, _OUT_DIM), lambda i: (0, 0)),
            pl.BlockSpec((1, _IN_DIM), lambda i: (0, 0)),
            pl.BlockSpec((8, _IN_DIM), lambda i: (0, 0)),
        ],
        out_specs=[
            pl.BlockSpec((_BLK_B, _OUT_DIM), lambda i, nb=nb: (nb - 1 - i, 0)),
            pl.BlockSpec(memory_space=pltpu.SMEM),
            pl.BlockSpec((8, _IN_DIM), lambda i: (0, 0)),
        ],
        out_shape=[
            jax.ShapeDtypeStruct((_MEM_LEN, _OUT_DIM), f32),
            jax.ShapeDtypeStruct((1, 1), f32),
            jax.ShapeDtypeStruct((_MEM_LEN, _IN_DIM), f32),
        ],
        scratch_shapes=[pltpu.SMEM((1,), f32)],
        input_output_aliases={3: 2},
    )(memory, enc, x, new_mem_data)

    return loss11[0, 0], new_memory, new_mem_data


# speedup vs baseline: 28.8880x; 1.0037x over previous
import jax
import jax.numpy as jnp
from jax.experimental import pallas as pl
from jax.experimental.pallas import tpu as pltpu

_IN_DIM = 256
_OUT_DIM = 512
_MEM_LEN = 100000
_BETA = 1.0
_BLK_A = 5000
_BLK_B = 5000


def _pass_a_body(in_ref, out_ref, sum_ref, sumsq_ref, acc_s, acc_q):
    i = pl.program_id(0)
    blk = in_ref[...]
    out_ref[...] = blk
    s = jnp.sum(blk, axis=0, keepdims=True)
    q = jnp.sum(blk * blk, axis=0, keepdims=True)

    @pl.when(i == 0)
    def _():
        acc_s[...] = s
        acc_q[...] = q

    @pl.when(i > 0)
    def _():
        acc_s[...] = acc_s[...] + s
        acc_q[...] = acc_q[...] + q

    @pl.when(i == pl.num_programs(0) - 1)
    def _():
        sum_ref[...] = acc_s[...]
        sumsq_ref[...] = acc_q[...]


def _mlp_body(x_ref, s_ref, q_ref, w1, b1, w2, b2, w3, b3, enc_ref):
    n = jnp.float32(_MEM_LEN)
    s = s_ref[...]
    q = q_ref[...]
    mean = s / n
    var = (q - s * (s / n)) / (n - 1.0)
    std = jnp.sqrt(var)
    xn = (x_ref[...] - mean) / std
    xn = jnp.where(std == 0.0, 0.0, xn)
    h1 = jnp.maximum(
        jnp.dot(xn, w1[...], preferred_element_type=jnp.float32) + b1[...], 0.0)
    h2 = jnp.maximum(
        jnp.dot(h1, w2[...], preferred_element_type=jnp.float32) + b2[...], 0.0)
    enc_ref[...] = jnp.tanh(
        jnp.dot(h2, w3[...], preferred_element_type=jnp.float32) + b3[...])


def _pass_b_body(mem_ref, enc_ref, x_ref, md_in, out_ref, loss_ref, md_out,
                 min_s):
    i = pl.program_id(0)
    blk = mem_ref[...]
    out_ref[...] = blk
    m = jnp.min(jnp.sum(jnp.abs(blk - enc_ref[...]), axis=1))

    @pl.when(i == 0)
    def _():
        min_s[0] = m

    @pl.when(i > 0)
    def _():
        min_s[0] = jnp.minimum(min_s[0], m)

    @pl.when(i == pl.num_programs(0) - 1)
    def _():
        loss = min_s[0]
        loss_ref[0, 0] = loss
        md_out[...] = md_in[...]

        @pl.when(loss <= _BETA)
        def _():
            out_ref[0:1, :] = enc_ref[...]
            md_out[0:1, :] = x_ref[...]


def kernel(x, mem_data, memory, W1, b1, W2, b2, W3, b3):
    f32 = jnp.float32
    W1p = jnp.pad(W1, ((0, 0), (0, 12)))
    b1p = jnp.pad(b1, (0, 12)).reshape(1, 512)
    W2p = jnp.pad(W2, ((0, 12), (0, 24)))
    b2p = jnp.pad(b2, (0, 24)).reshape(1, 1024)
    W3p = jnp.pad(W3, ((0, 24), (0, 0)))
    b3p = b3.reshape(1, 512)

    na = _MEM_LEN // _BLK_A
    new_mem_data, col_sum, col_sumsq = pl.pallas_call(
        _pass_a_body,
        grid=(na,),
        in_specs=[pl.BlockSpec((_BLK_A, _IN_DIM), lambda i: (i, 0))],
        out_specs=[
            pl.BlockSpec((_BLK_A, _IN_DIM), lambda i: (i, 0)),
            pl.BlockSpec((1, _IN_DIM), lambda i: (0, 0)),
            pl.BlockSpec((1, _IN_DIM), lambda i: (0, 0)),
        ],
        out_shape=[
            jax.ShapeDtypeStruct((_MEM_LEN, _IN_DIM), f32),
            jax.ShapeDtypeStruct((1, _IN_DIM), f32),
            jax.ShapeDtypeStruct((1, _IN_DIM), f32),
        ],
        scratch_shapes=[
            pltpu.VMEM((1, _IN_DIM), f32),
            pltpu.VMEM((1, _IN_DIM), f32),
        ],
    )(mem_data)

    enc = pl.pallas_call(
        _mlp_body,
        out_shape=jax.ShapeDtypeStruct((1, _OUT_DIM), f32),
    )(x, col_sum, col_sumsq, W1p, b1p, W2p, b2p, W3p, b3p)

    nb = _MEM_LEN // _BLK_B
    new_memory, loss11, new_mem_data = pl.pallas_call(
        _pass_b_body,
        grid=(nb,),
        in_specs=[
            pl.BlockSpec((_BLK_B, _OUT_DIM), lambda i, nb=nb: (nb - 1 - i, 0)),
            pl.BlockSpec((1, _OUT_DIM), lambda i: (0, 0)),
            pl.BlockSpec((1, _IN_DIM), lambda i: (0, 0)),
            pl.BlockSpec((8, _IN_DIM), lambda i: (0, 0)),
        ],
        out_specs=[
            pl.BlockSpec((_BLK_B, _OUT_DIM), lambda i, nb=nb: (nb - 1 - i, 0)),
            pl.BlockSpec(memory_space=pltpu.SMEM),
            pl.BlockSpec((8, _IN_DIM), lambda i: (0, 0)),
        ],
        out_shape=[
            jax.ShapeDtypeStruct((_MEM_LEN, _OUT_DIM), f32),
            jax.ShapeDtypeStruct((1, 1), f32),
            jax.ShapeDtypeStruct((_MEM_LEN, _IN_DIM), f32),
        ],
        scratch_shapes=[pltpu.SMEM((1,), f32)],
        input_output_aliases={3: 2},
    )(memory, enc, x, new_mem_data)

    return loss11[0, 0], new_memory, new_mem_data
